# baseline (speedup 1.0000x reference)
import jax
import jax.numpy as jnp
from jax import lax
from jax.experimental import pallas as pl
from jax.experimental.pallas import tpu as pltpu
from jax.experimental.pallas import tpu_sc as plsc

_N_HEADS = 8
_EMBED = 256
_HEAD_DIM = 32
_B = 16
_LQ = 300
_LV = 8500
_BQ = _B * _LQ
_ROWS = _BQ * _N_HEADS

_NW = 32
_BQ_PER_W = _BQ // _NW
_ROW0_STEP = _BQ_PER_W * _N_HEADS


def _mm_kernel(x_ref, w_ref, b_ref, o_ref):
    o_ref[...] = (
        jnp.dot(x_ref[...], w_ref[...], preferred_element_type=jnp.float32)
        + b_ref[...]
    )


def _matmul_bias(x, w, b, bm):
    m, k = x.shape
    n = w.shape[1]
    return pl.pallas_call(
        _mm_kernel,
        grid=(m // bm,),
        in_specs=[
            pl.BlockSpec((bm, k), lambda i: (i, 0)),
            pl.BlockSpec((k, n), lambda i: (0, 0)),
            pl.BlockSpec((1, n), lambda i: (0, 0)),
        ],
        out_specs=pl.BlockSpec((bm, n), lambda i: (i, 0)),
        out_shape=jax.ShapeDtypeStruct((m, n), jnp.float32),
    )(x, w, b.reshape(1, n))


def _splat(val):
    return jnp.full((16,), val)


def _sc_body(s_hbm, rp_hbm, v_hbm, out_hbm, s_buf, rp_buf, idx_buf, w_buf,
             g_buf, out_buf, gsem):
    nc = 2
    wid = lax.axis_index("s") * nc + lax.axis_index("c")
    b = wid // 2
    vbase = b * (_LV * _N_HEADS)

    iota = lax.iota(jnp.int32, 16)
    level = lax.shift_right_logical(iota, 2)
    wpi = lax.shift_right_logical(jnp.full((16,), 80, jnp.int32), level)
    offp = jnp.where(
        level == 0, 0,
        jnp.where(level == 1, 6400, jnp.where(level == 2, 8000, 8400)))
    wp = wpi.astype(jnp.float32)
    hp = wp
    wm1 = wpi - 1
    hm1 = wm1

    bq0 = wid * _BQ_PER_W
    pltpu.sync_copy(s_hbm.at[bq0], s_buf.at[0])
    pltpu.sync_copy(rp_hbm.at[bq0 // 4], rp_buf.at[0])

    def body(i, carry):
        par = lax.rem(i, 2)

        @pl.when(i < _BQ_PER_W)
        def _produce():
            bq = wid * _BQ_PER_W + i
            u = (bq % 4) * 4
            rx = plsc.load_gather(rp_buf, [_splat(par), _splat(u)])
            ry = plsc.load_gather(rp_buf, [_splat(par), _splat(u + 1)])
            rw = plsc.load_gather(rp_buf, [_splat(par), _splat(u + 2)])
            rh = plsc.load_gather(rp_buf, [_splat(par), _splat(u + 3)])
            sclx = rw * 0.125
            scly = rh * 0.125

            for h in range(_N_HEADS):
                logits = s_buf[par, pl.ds(_EMBED + h * 16, 16)]
                mx = jnp.max(logits)
                e = jnp.exp(logits - mx)
                p = e / jnp.full((16,), jnp.sum(e))

                sx = plsc.load_gather(
                    s_buf, [_splat(par), iota * 2 + (h * 32)])
                sy = plsc.load_gather(
                    s_buf, [_splat(par), iota * 2 + (h * 32 + 1)])
                x = (rx + sx * sclx) * wp - 0.5
                y = (ry + sy * scly) * hp - 0.5
                xi = x.astype(jnp.int32)
                xf = xi.astype(jnp.float32)
                x0i = jnp.where(xf > x, xi - 1, xi)
                fx = x - jnp.where(xf > x, xf - 1.0, xf)
                yi = y.astype(jnp.int32)
                yf = yi.astype(jnp.float32)
                y0i = jnp.where(yf > y, yi - 1, yi)
                fy = y - jnp.where(yf > y, yf - 1.0, yf)
                gx = 1.0 - fx
                gy = 1.0 - fy
                x1i = x0i + 1
                y1i = y0i + 1
                vx0 = (x0i >= 0) & (x0i <= wm1)
                vx1 = (x1i >= 0) & (x1i <= wm1)
                vy0 = (y0i >= 0) & (y0i <= hm1)
                vy1 = (y1i >= 0) & (y1i <= hm1)
                cx0 = jnp.clip(x0i, 0, wm1)
                cx1 = jnp.clip(x1i, 0, wm1)
                cy0 = jnp.clip(y0i, 0, hm1)
                cy1 = jnp.clip(y1i, 0, hm1)
                r0 = offp + cy0 * wpi
                r1 = offp + cy1 * wpi
                base = vbase + h
                corners = (
                    (r0 + cx0, gx * gy * (vx0 & vy0).astype(jnp.float32)),
                    (r0 + cx1, fx * gy * (vx1 & vy0).astype(jnp.float32)),
                    (r1 + cx0, gx * fy * (vx0 & vy1).astype(jnp.float32)),
                    (r1 + cx1, fx * fy * (vx1 & vy1).astype(jnp.float32)),
                )
                for c, (ridx, wgt) in enumerate(corners):
                    flat = h * 64 + c * 16
                    idx_buf[par, flat >> 7, pl.ds(flat & 127, 16)] = (
                        ridx * 8 + base)
                    w_buf[par, pl.ds(flat, 16)] = wgt * p

            for j in range(4):
                pltpu.async_copy(
                    v_hbm.at[idx_buf.at[par, j]],
                    g_buf.at[par, pl.ds(j * 128, 128)],
                    gsem.at[par],
                )

            bqn = wid * _BQ_PER_W + jnp.minimum(i + 1, _BQ_PER_W - 1)
            pltpu.sync_copy(s_hbm.at[bqn], s_buf.at[1 - par])
            pltpu.sync_copy(rp_hbm.at[bqn // 4], rp_buf.at[1 - par])

        @pl.when(i > 0)
        def _consume():
            i1 = i - 1
            par1 = lax.rem(i1, 2)
            for _ in range(4):
                pltpu.make_async_copy(
                    v_hbm.at[idx_buf.at[par1, 0]],
                    g_buf.at[par1, pl.ds(0, 128)],
                    gsem.at[par1],
                ).wait()

            def acc_row(r, carry2):
                acc0 = jnp.zeros((16,), jnp.float32)
                acc1 = jnp.zeros((16,), jnp.float32)
                for j in range(64):
                    n_ = r * 64 + j
                    wj = plsc.load_gather(
                        w_buf, [_splat(par1), _splat(n_)])
                    acc0 = acc0 + wj * g_buf[par1, n_, pl.ds(0, 16)]
                    acc1 = acc1 + wj * g_buf[par1, n_, pl.ds(16, 16)]
                out_buf[r, pl.ds(0, 16)] = acc0
                out_buf[r, pl.ds(16, 16)] = acc1
                return carry2

            lax.fori_loop(0, _N_HEADS, acc_row, 0)

            row0 = wid * _ROW0_STEP + i1 * _N_HEADS
            pltpu.sync_copy(out_buf, out_hbm.at[pl.ds(row0, _N_HEADS)])

        return carry

    lax.fori_loop(0, _BQ_PER_W + 1, body, 0)


def _sc_sample(s_all, rp16, vrows):
    mesh = plsc.VectorSubcoreMesh(core_axis_name="c", subcore_axis_name="s")
    f = pl.kernel(
        _sc_body,
        out_type=jax.ShapeDtypeStruct((_ROWS, _HEAD_DIM), jnp.float32),
        mesh=mesh,
        compiler_params=pltpu.CompilerParams(
            needs_layout_passes=False, use_tc_tiling_on_sc=False),
        scratch_types=[
            pltpu.VMEM((2, 384), jnp.float32),
            pltpu.VMEM((2, 16), jnp.float32),
            pltpu.VMEM((2, 4, 128), jnp.int32),
            pltpu.VMEM((2, 512), jnp.float32),
            pltpu.VMEM((2, 512, _HEAD_DIM), jnp.float32),
            pltpu.VMEM((_N_HEADS, _HEAD_DIM), jnp.float32),
            pltpu.SemaphoreType.DMA((2,)),
        ],
    )
    return f(s_all, rp16, vrows)


def kernel(query, reference_points, value, Wv, bv, Ws, bs_, Wa, ba, Wo, bo):
    q2 = query.reshape(_BQ, _EMBED)
    v2 = value.reshape(_B * _LV, _EMBED)

    vmat = _matmul_bias(v2, Wv, bv, bm=544)
    wsa = jnp.concatenate([Ws, Wa], axis=1)
    bsa = jnp.concatenate([bs_, ba])
    s_all = _matmul_bias(q2, wsa, bsa, bm=600)

    rp16 = reference_points.reshape(_BQ // 4, 16)
    vrows = vmat.reshape(_B * _LV * _N_HEADS, _HEAD_DIM)
    sampled = _sc_sample(s_all, rp16, vrows)

    out = _matmul_bias(sampled.reshape(_BQ, _EMBED), Wo, bo, bm=600)
    return out.reshape(_B, _LQ, _EMBED)


# speedup vs baseline: 1.4578x; 1.4578x over previous
import jax
import jax.numpy as jnp
from jax import lax
from jax.experimental import pallas as pl
from jax.experimental.pallas import tpu as pltpu
from jax.experimental.pallas import tpu_sc as plsc

_N_HEADS = 8
_EMBED = 256
_HEAD_DIM = 32
_B = 16
_LQ = 300
_LV = 8500
_BQ = _B * _LQ
_ROWS = _BQ * _N_HEADS

_NW = 32
_BQ_PER_W = _BQ // _NW
_ROW0_STEP = _BQ_PER_W * _N_HEADS


def _mm_kernel(x_ref, w_ref, b_ref, o_ref):
    o_ref[...] = (
        jnp.dot(x_ref[...], w_ref[...], preferred_element_type=jnp.float32)
        + b_ref[...]
    )


def _mm2_kernel(x_ref, w_ref, b_ref, o1_ref, o2_ref):
    res = (
        jnp.dot(x_ref[...], w_ref[...], preferred_element_type=jnp.float32)
        + b_ref[...]
    )
    o1_ref[...] = res[:, :128]
    o2_ref[...] = res[:, 128:]


def _matmul_bias_split(x, w, b, bm):
    m, k = x.shape
    n = w.shape[1]
    out1, out2 = pl.pallas_call(
        _mm2_kernel,
        grid=(m // bm,),
        in_specs=[
            pl.BlockSpec((bm, k), lambda i: (i, 0)),
            pl.BlockSpec((k, n), lambda i: (0, 0)),
            pl.BlockSpec((1, n), lambda i: (0, 0)),
        ],
        out_specs=[
            pl.BlockSpec((bm, 128), lambda i: (i, 0)),
            pl.BlockSpec((bm, 128), lambda i: (i, 0)),
        ],
        out_shape=[
            jax.ShapeDtypeStruct((m, 128), jnp.float32),
            jax.ShapeDtypeStruct((m, 128), jnp.float32),
        ],
    )(x, w, b.reshape(1, n))
    return out1, out2


def _matmul_bias(x, w, b, bm):
    m, k = x.shape
    n = w.shape[1]
    return pl.pallas_call(
        _mm_kernel,
        grid=(m // bm,),
        in_specs=[
            pl.BlockSpec((bm, k), lambda i: (i, 0)),
            pl.BlockSpec((k, n), lambda i: (0, 0)),
            pl.BlockSpec((1, n), lambda i: (0, 0)),
        ],
        out_specs=pl.BlockSpec((bm, n), lambda i: (i, 0)),
        out_shape=jax.ShapeDtypeStruct((m, n), jnp.float32),
    )(x, w, b.reshape(1, n))


def _splat(val):
    return jnp.full((16,), val)


def _sc_body(s_hbm, rp_hbm, ve_hbm, vo_hbm, out_hbm, s_buf, rp_buf, idx_buf,
             w_buf, g_buf, out_buf, gsem):
    nc = 2
    wid = lax.axis_index("s") * nc + lax.axis_index("c")
    b = wid // 2
    vbase = b * 4

    iota = lax.iota(jnp.int32, 16)
    level = lax.shift_right_logical(iota, 2)
    wpi = lax.shift_right_logical(jnp.full((16,), 80, jnp.int32), level)
    offp = jnp.where(
        level == 0, 0,
        jnp.where(level == 1, 6400, jnp.where(level == 2, 8000, 8400)))
    wp = wpi.astype(jnp.float32)
    hp = wp
    wm1 = wpi - 1
    hm1 = wm1

    bq0 = wid * _BQ_PER_W
    pltpu.sync_copy(s_hbm.at[bq0], s_buf.at[0])
    pltpu.sync_copy(rp_hbm.at[bq0 // 4], rp_buf.at[0])

    def body(i, carry):
        par = lax.rem(i, 2)

        @pl.when(i < _BQ_PER_W)
        def _produce():
            bq = wid * _BQ_PER_W + i
            u = (bq % 4) * 4
            rx = plsc.load_gather(rp_buf, [_splat(par), _splat(u)])
            ry = plsc.load_gather(rp_buf, [_splat(par), _splat(u + 1)])
            rw = plsc.load_gather(rp_buf, [_splat(par), _splat(u + 2)])
            rh = plsc.load_gather(rp_buf, [_splat(par), _splat(u + 3)])
            sclx = rw * 0.125
            scly = rh * 0.125

            for h in range(_N_HEADS):
                logits = s_buf[par, pl.ds(_EMBED + h * 16, 16)]
                mx = jnp.max(logits)
                e = jnp.exp(logits - mx)
                p = e / jnp.full((16,), jnp.sum(e))

                sx = plsc.load_gather(
                    s_buf, [_splat(par), iota * 2 + (h * 32)])
                sy = plsc.load_gather(
                    s_buf, [_splat(par), iota * 2 + (h * 32 + 1)])
                x = (rx + sx * sclx) * wp - 0.5
                y = (ry + sy * scly) * hp - 0.5
                xi = x.astype(jnp.int32)
                xf = xi.astype(jnp.float32)
                x0i = jnp.where(xf > x, xi - 1, xi)
                fx = x - jnp.where(xf > x, xf - 1.0, xf)
                yi = y.astype(jnp.int32)
                yf = yi.astype(jnp.float32)
                y0i = jnp.where(yf > y, yi - 1, yi)
                fy = y - jnp.where(yf > y, yf - 1.0, yf)
                gx = 1.0 - fx
                gy = 1.0 - fy
                x1i = x0i + 1
                y1i = y0i + 1
                vx0 = (x0i >= 0) & (x0i <= wm1)
                vx1 = (x1i >= 0) & (x1i <= wm1)
                vy0 = (y0i >= 0) & (y0i <= hm1)
                vy1 = (y1i >= 0) & (y1i <= hm1)
                cx0 = jnp.clip(x0i, 0, wm1)
                cx1 = jnp.clip(x1i, 0, wm1)
                cy0 = jnp.clip(y0i, 0, hm1)
                cy1 = jnp.clip(y1i, 0, hm1)
                r0 = offp + cy0 * wpi
                r1 = offp + cy1 * wpi
                base = vbase + (h & 3)
                corners = (
                    (r0 + cx0, gx * gy * (vx0 & vy0).astype(jnp.float32)),
                    (r0 + cx1, fx * gy * (vx1 & vy0).astype(jnp.float32)),
                    (r1 + cx0, gx * fy * (vx0 & vy1).astype(jnp.float32)),
                    (r1 + cx1, fx * fy * (vx1 & vy1).astype(jnp.float32)),
                )
                for c, (ridx, wgt) in enumerate(corners):
                    flat = h * 64 + c * 16
                    idx_buf[par, flat >> 7, pl.ds(flat & 127, 16)] = (
                        ridx * 64 + base)
                    w_buf[par, pl.ds(flat, 16)] = wgt * p

            for j in range(4):
                tbl = ve_hbm if j < 2 else vo_hbm
                pltpu.async_copy(
                    tbl.at[idx_buf.at[par, j]],
                    g_buf.at[par, pl.ds(j * 128, 128)],
                    gsem.at[par],
                )

            bqn = wid * _BQ_PER_W + jnp.minimum(i + 1, _BQ_PER_W - 1)
            pltpu.sync_copy(s_hbm.at[bqn], s_buf.at[1 - par])
            pltpu.sync_copy(rp_hbm.at[bqn // 4], rp_buf.at[1 - par])

        @pl.when(i > 0)
        def _consume():
            i1 = i - 1
            par1 = lax.rem(i1, 2)
            for _ in range(4):
                pltpu.make_async_copy(
                    ve_hbm.at[idx_buf.at[par1, 0]],
                    g_buf.at[par1, pl.ds(0, 128)],
                    gsem.at[par1],
                ).wait()

            def acc_row(r, carry2):
                acc0 = jnp.zeros((16,), jnp.float32)
                acc1 = jnp.zeros((16,), jnp.float32)
                for j in range(64):
                    n_ = r * 64 + j
                    wj = plsc.load_gather(
                        w_buf, [_splat(par1), _splat(n_)])
                    acc0 = acc0 + wj * g_buf[par1, n_, pl.ds(0, 16)]
                    acc1 = acc1 + wj * g_buf[par1, n_, pl.ds(16, 16)]
                out_buf[r, pl.ds(0, 16)] = acc0
                out_buf[r, pl.ds(16, 16)] = acc1
                return carry2

            lax.fori_loop(0, _N_HEADS, acc_row, 0)

            row0 = wid * _ROW0_STEP + i1 * _N_HEADS
            pltpu.sync_copy(out_buf, out_hbm.at[pl.ds(row0, _N_HEADS)])

        return carry

    lax.fori_loop(0, _BQ_PER_W + 1, body, 0)


def _sc_sample(s_all, rp16, veven, vodd):
    mesh = plsc.VectorSubcoreMesh(core_axis_name="c", subcore_axis_name="s")
    f = pl.kernel(
        _sc_body,
        out_type=jax.ShapeDtypeStruct((_ROWS, _HEAD_DIM), jnp.float32),
        mesh=mesh,
        compiler_params=pltpu.CompilerParams(
            needs_layout_passes=False, use_tc_tiling_on_sc=False),
        scratch_types=[
            pltpu.VMEM((2, 384), jnp.float32),
            pltpu.VMEM((2, 16), jnp.float32),
            pltpu.VMEM((2, 4, 128), jnp.int32),
            pltpu.VMEM((2, 512), jnp.float32),
            pltpu.VMEM((2, 512, _HEAD_DIM), jnp.float32),
            pltpu.VMEM((_N_HEADS, _HEAD_DIM), jnp.float32),
            pltpu.SemaphoreType.DMA((2,)),
        ],
    )
    return f(s_all, rp16, veven, vodd)


def kernel(query, reference_points, value, Wv, bv, Ws, bs_, Wa, ba, Wo, bo):
    q2 = query.reshape(_BQ, _EMBED)
    v2 = value.transpose(1, 0, 2).reshape(_LV * _B, _EMBED)

    ve, vo = _matmul_bias_split(v2, Wv, bv, bm=544)
    wsa = jnp.concatenate([Ws, Wa], axis=1)
    bsa = jnp.concatenate([bs_, ba])
    s_all = _matmul_bias(q2, wsa, bsa, bm=600)

    rp16 = reference_points.reshape(_BQ // 4, 16)
    veven = ve.reshape(_LV * _B * 4, _HEAD_DIM)
    vodd = vo.reshape(_LV * _B * 4, _HEAD_DIM)
    sampled = _sc_sample(s_all, rp16, veven, vodd)

    out = _matmul_bias(sampled.reshape(_BQ, _EMBED), Wo, bo, bm=600)
    return out.reshape(_B, _LQ, _EMBED)


# speedup vs baseline: 1.8788x; 1.2888x over previous
import jax
import jax.numpy as jnp
from jax import lax
from jax.experimental import pallas as pl
from jax.experimental.pallas import tpu as pltpu
from jax.experimental.pallas import tpu_sc as plsc

_N_HEADS = 8
_EMBED = 256
_HEAD_DIM = 32
_B = 16
_LQ = 300
_LV = 8500
_BQ = _B * _LQ
_ROWS = _BQ * _N_HEADS

_NW = 32
_BQ_PER_W = _BQ // _NW
_ROW0_STEP = _BQ_PER_W * _N_HEADS


def _mm_kernel(x_ref, w_ref, b_ref, o_ref):
    o_ref[...] = (
        jnp.dot(x_ref[...], w_ref[...], preferred_element_type=jnp.float32)
        + b_ref[...]
    )


def _mm2_kernel(x_ref, w_ref, b_ref, o1_ref, o2_ref):
    res = (
        jnp.dot(x_ref[...], w_ref[...], preferred_element_type=jnp.float32)
        + b_ref[...]
    )
    o1_ref[...] = res[:, :128]
    o2_ref[...] = res[:, 128:]


def _matmul_bias_split(x, w, b, bm):
    m, k = x.shape
    n = w.shape[1]
    out1, out2 = pl.pallas_call(
        _mm2_kernel,
        grid=(m // bm,),
        in_specs=[
            pl.BlockSpec((bm, k), lambda i: (i, 0)),
            pl.BlockSpec((k, n), lambda i: (0, 0)),
            pl.BlockSpec((1, n), lambda i: (0, 0)),
        ],
        out_specs=[
            pl.BlockSpec((bm, 128), lambda i: (i, 0)),
            pl.BlockSpec((bm, 128), lambda i: (i, 0)),
        ],
        out_shape=[
            jax.ShapeDtypeStruct((m, 128), jnp.float32),
            jax.ShapeDtypeStruct((m, 128), jnp.float32),
        ],
    )(x, w, b.reshape(1, n))
    return out1, out2


def _matmul_bias(x, w, b, bm):
    m, k = x.shape
    n = w.shape[1]
    return pl.pallas_call(
        _mm_kernel,
        grid=(m // bm,),
        in_specs=[
            pl.BlockSpec((bm, k), lambda i: (i, 0)),
            pl.BlockSpec((k, n), lambda i: (0, 0)),
            pl.BlockSpec((1, n), lambda i: (0, 0)),
        ],
        out_specs=pl.BlockSpec((bm, n), lambda i: (i, 0)),
        out_shape=jax.ShapeDtypeStruct((m, n), jnp.float32),
    )(x, w, b.reshape(1, n))


def _splat(val):
    return jnp.full((16,), val)


def _sc_body(s_hbm, rp_hbm, ve_hbm, vo_hbm, out_hbm, s_buf, rp_buf, idx_buf,
             w_buf, g_buf, out_buf, gsem, ssem, osem):
    nc = 2
    wid = lax.axis_index("s") * nc + lax.axis_index("c")
    b = wid // 2
    vbase = b * 4

    iota = lax.iota(jnp.int32, 16)
    level = lax.shift_right_logical(iota, 2)
    wpi = lax.shift_right_logical(jnp.full((16,), 80, jnp.int32), level)
    offp = jnp.where(
        level == 0, 0,
        jnp.where(level == 1, 6400, jnp.where(level == 2, 8000, 8400)))
    wp = wpi.astype(jnp.float32)
    hp = wp
    wm1 = wpi - 1
    hm1 = wm1

    bq0 = wid * _BQ_PER_W
    pltpu.async_copy(s_hbm.at[bq0], s_buf.at[0], ssem.at[0])
    pltpu.async_copy(rp_hbm.at[bq0 // 4], rp_buf.at[0], ssem.at[0])

    def body(i, carry):
        par = lax.rem(i, 2)

        @pl.when(i < _BQ_PER_W)
        def _produce():
            bq = wid * _BQ_PER_W + i
            bqn = wid * _BQ_PER_W + jnp.minimum(i + 1, _BQ_PER_W - 1)
            pltpu.async_copy(s_hbm.at[bqn], s_buf.at[1 - par],
                             ssem.at[1 - par])
            pltpu.async_copy(rp_hbm.at[bqn // 4], rp_buf.at[1 - par],
                             ssem.at[1 - par])
            pltpu.make_async_copy(
                s_hbm.at[bq], s_buf.at[par], ssem.at[par]).wait()
            pltpu.make_async_copy(
                rp_hbm.at[bq // 4], rp_buf.at[par], ssem.at[par]).wait()
            u = (bq % 4) * 4
            rx = plsc.load_gather(rp_buf, [_splat(par), _splat(u)])
            ry = plsc.load_gather(rp_buf, [_splat(par), _splat(u + 1)])
            rw = plsc.load_gather(rp_buf, [_splat(par), _splat(u + 2)])
            rh = plsc.load_gather(rp_buf, [_splat(par), _splat(u + 3)])
            sclx = rw * 0.125
            scly = rh * 0.125

            for h in range(_N_HEADS):
                logits = s_buf[par, pl.ds(_EMBED + h * 16, 16)]
                mx = jnp.max(logits)
                e = jnp.exp(logits - mx)
                p = e / jnp.full((16,), jnp.sum(e))

                sx = plsc.load_gather(
                    s_buf, [_splat(par), iota * 2 + (h * 32)])
                sy = plsc.load_gather(
                    s_buf, [_splat(par), iota * 2 + (h * 32 + 1)])
                x = (rx + sx * sclx) * wp - 0.5
                y = (ry + sy * scly) * hp - 0.5
                xi = x.astype(jnp.int32)
                xf = xi.astype(jnp.float32)
                x0i = jnp.where(xf > x, xi - 1, xi)
                fx = x - jnp.where(xf > x, xf - 1.0, xf)
                yi = y.astype(jnp.int32)
                yf = yi.astype(jnp.float32)
                y0i = jnp.where(yf > y, yi - 1, yi)
                fy = y - jnp.where(yf > y, yf - 1.0, yf)
                gx = 1.0 - fx
                gy = 1.0 - fy
                x1i = x0i + 1
                y1i = y0i + 1
                vx0 = (x0i >= 0) & (x0i <= wm1)
                vx1 = (x1i >= 0) & (x1i <= wm1)
                vy0 = (y0i >= 0) & (y0i <= hm1)
                vy1 = (y1i >= 0) & (y1i <= hm1)
                cx0 = jnp.clip(x0i, 0, wm1)
                cx1 = jnp.clip(x1i, 0, wm1)
                cy0 = jnp.clip(y0i, 0, hm1)
                cy1 = jnp.clip(y1i, 0, hm1)
                r0 = offp + cy0 * wpi
                r1 = offp + cy1 * wpi
                base = vbase + (h & 3)
                corners = (
                    (r0 + cx0, gx * gy * (vx0 & vy0).astype(jnp.float32)),
                    (r0 + cx1, fx * gy * (vx1 & vy0).astype(jnp.float32)),
                    (r1 + cx0, gx * fy * (vx0 & vy1).astype(jnp.float32)),
                    (r1 + cx1, fx * fy * (vx1 & vy1).astype(jnp.float32)),
                )
                for c, (ridx, wgt) in enumerate(corners):
                    flat = h * 64 + c * 16
                    idx_buf[par, flat >> 7, pl.ds(flat & 127, 16)] = (
                        ridx * 64 + base)
                    w_buf[par, pl.ds(flat, 16)] = wgt * p

            for j in range(4):
                tbl = ve_hbm if j < 2 else vo_hbm
                pltpu.async_copy(
                    tbl.at[idx_buf.at[par, j]],
                    g_buf.at[par, pl.ds(j * 128, 128)],
                    gsem.at[par],
                )


        @pl.when(i > 0)
        def _consume():
            i1 = i - 1
            par1 = lax.rem(i1, 2)
            for _ in range(4):
                pltpu.make_async_copy(
                    ve_hbm.at[idx_buf.at[par1, 0]],
                    g_buf.at[par1, pl.ds(0, 128)],
                    gsem.at[par1],
                ).wait()

            row0 = wid * _ROW0_STEP + i1 * _N_HEADS

            @pl.when(i1 >= 2)
            def _drain_out():
                pltpu.make_async_copy(
                    out_buf.at[par1],
                    out_hbm.at[pl.ds(row0, _N_HEADS)],
                    osem.at[par1]).wait()

            def acc_row(r, carry2):
                acc0 = jnp.zeros((16,), jnp.float32)
                acc1 = jnp.zeros((16,), jnp.float32)
                for j in range(64):
                    n_ = r * 64 + j
                    wj = plsc.load_gather(
                        w_buf, [_splat(par1), _splat(n_)])
                    acc0 = acc0 + wj * g_buf[par1, n_, pl.ds(0, 16)]
                    acc1 = acc1 + wj * g_buf[par1, n_, pl.ds(16, 16)]
                out_buf[par1, r, pl.ds(0, 16)] = acc0
                out_buf[par1, r, pl.ds(16, 16)] = acc1
                return carry2

            lax.fori_loop(0, _N_HEADS, acc_row, 0)

            pltpu.async_copy(
                out_buf.at[par1],
                out_hbm.at[pl.ds(row0, _N_HEADS)],
                osem.at[par1])

        return carry

    lax.fori_loop(0, _BQ_PER_W + 1, body, 0)

    for parf in range(2):
        pltpu.make_async_copy(
            out_buf.at[parf],
            out_hbm.at[pl.ds(wid * _ROW0_STEP, _N_HEADS)],
            osem.at[parf]).wait()


def _sc_sample(s_all, rp16, veven, vodd):
    mesh = plsc.VectorSubcoreMesh(core_axis_name="c", subcore_axis_name="s")
    f = pl.kernel(
        _sc_body,
        out_type=jax.ShapeDtypeStruct((_ROWS, _HEAD_DIM), jnp.float32),
        mesh=mesh,
        compiler_params=pltpu.CompilerParams(
            needs_layout_passes=False, use_tc_tiling_on_sc=False),
        scratch_types=[
            pltpu.VMEM((2, 384), jnp.float32),
            pltpu.VMEM((2, 16), jnp.float32),
            pltpu.VMEM((2, 4, 128), jnp.int32),
            pltpu.VMEM((2, 512), jnp.float32),
            pltpu.VMEM((2, 512, _HEAD_DIM), jnp.float32),
            pltpu.VMEM((2, _N_HEADS, _HEAD_DIM), jnp.float32),
            pltpu.SemaphoreType.DMA((2,)),
            pltpu.SemaphoreType.DMA((2,)),
            pltpu.SemaphoreType.DMA((2,)),
        ],
    )
    return f(s_all, rp16, veven, vodd)


def kernel(query, reference_points, value, Wv, bv, Ws, bs_, Wa, ba, Wo, bo):
    q2 = query.reshape(_BQ, _EMBED)
    v2 = value.transpose(1, 0, 2).reshape(_LV * _B, _EMBED)

    ve, vo = _matmul_bias_split(v2, Wv, bv, bm=544)
    wsa = jnp.concatenate([Ws, Wa], axis=1)
    bsa = jnp.concatenate([bs_, ba])
    s_all = _matmul_bias(q2, wsa, bsa, bm=600)

    rp16 = reference_points.reshape(_BQ // 4, 16)
    veven = ve.reshape(_LV * _B * 4, _HEAD_DIM)
    vodd = vo.reshape(_LV * _B * 4, _HEAD_DIM)
    sampled = _sc_sample(s_all, rp16, veven, vodd)

    out = _matmul_bias(sampled.reshape(_BQ, _EMBED), Wo, bo, bm=600)
    return out.reshape(_B, _LQ, _EMBED)


# speedup vs baseline: 2.1539x; 1.1464x over previous
import jax
import jax.numpy as jnp
from jax import lax
from jax.experimental import pallas as pl
from jax.experimental.pallas import tpu as pltpu
from jax.experimental.pallas import tpu_sc as plsc

_N_HEADS = 8
_EMBED = 256
_HEAD_DIM = 32
_B = 16
_LQ = 300
_LV = 8500
_BQ = _B * _LQ
_ROWS = _BQ * _N_HEADS

_NW = 32
_BQ_PER_W = _BQ // _NW
_ROW0_STEP = _BQ_PER_W * _N_HEADS


def _mm_kernel(x_ref, w_ref, b_ref, o_ref):
    o_ref[...] = (
        jnp.dot(x_ref[...], w_ref[...], preferred_element_type=jnp.float32)
        + b_ref[...]
    )


def _mm2_kernel(x_ref, w_ref, b_ref, o1_ref, o2_ref):
    res = (
        jnp.dot(x_ref[...], w_ref[...], preferred_element_type=jnp.float32,
                precision=lax.Precision.DEFAULT)
        + b_ref[...]
    )
    o1_ref[...] = res[:, :128]
    o2_ref[...] = res[:, 128:]


def _matmul_bias_split(x, w, b, bm):
    m, k = x.shape
    n = w.shape[1]
    out1, out2 = pl.pallas_call(
        _mm2_kernel,
        grid=(m // bm,),
        in_specs=[
            pl.BlockSpec((bm, k), lambda i: (i, 0)),
            pl.BlockSpec((k, n), lambda i: (0, 0)),
            pl.BlockSpec((1, n), lambda i: (0, 0)),
        ],
        out_specs=[
            pl.BlockSpec((bm, 128), lambda i: (i, 0)),
            pl.BlockSpec((bm, 128), lambda i: (i, 0)),
        ],
        out_shape=[
            jax.ShapeDtypeStruct((m, 128), jnp.float32),
            jax.ShapeDtypeStruct((m, 128), jnp.float32),
        ],
    )(x, w, b.reshape(1, n))
    return out1, out2


def _matmul_bias(x, w, b, bm):
    m, k = x.shape
    n = w.shape[1]
    return pl.pallas_call(
        _mm_kernel,
        grid=(m // bm,),
        in_specs=[
            pl.BlockSpec((bm, k), lambda i: (i, 0)),
            pl.BlockSpec((k, n), lambda i: (0, 0)),
            pl.BlockSpec((1, n), lambda i: (0, 0)),
        ],
        out_specs=pl.BlockSpec((bm, n), lambda i: (i, 0)),
        out_shape=jax.ShapeDtypeStruct((m, n), jnp.float32),
    )(x, w, b.reshape(1, n))


def _splat(val):
    return jnp.full((16,), val)


def _sc_body(s_hbm, rp_hbm, ve_hbm, vo_hbm, out_hbm, s_buf, rp_buf, idx_buf,
             w_buf, g_buf, out_buf, gsem, ssem, osem):
    nc = 2
    wid = lax.axis_index("s") * nc + lax.axis_index("c")
    b = wid // 2
    vbase = b * 4

    iota = lax.iota(jnp.int32, 16)
    level = lax.shift_right_logical(iota, 2)
    wpi = lax.shift_right_logical(jnp.full((16,), 80, jnp.int32), level)
    offp = jnp.where(
        level == 0, 0,
        jnp.where(level == 1, 6400, jnp.where(level == 2, 8000, 8400)))
    wp = wpi.astype(jnp.float32)
    hp = wp
    wm1 = wpi - 1
    hm1 = wm1

    bq0 = wid * _BQ_PER_W
    pltpu.async_copy(s_hbm.at[bq0], s_buf.at[0], ssem.at[0])
    pltpu.async_copy(rp_hbm.at[bq0 // 4], rp_buf.at[0], ssem.at[0])

    def body(i, carry):
        par = lax.rem(i, 2)

        @pl.when(i < _BQ_PER_W)
        def _produce():
            bq = wid * _BQ_PER_W + i
            bqn = wid * _BQ_PER_W + jnp.minimum(i + 1, _BQ_PER_W - 1)
            pltpu.async_copy(s_hbm.at[bqn], s_buf.at[1 - par],
                             ssem.at[1 - par])
            pltpu.async_copy(rp_hbm.at[bqn // 4], rp_buf.at[1 - par],
                             ssem.at[1 - par])
            pltpu.make_async_copy(
                s_hbm.at[bq], s_buf.at[par], ssem.at[par]).wait()
            pltpu.make_async_copy(
                rp_hbm.at[bq // 4], rp_buf.at[par], ssem.at[par]).wait()
            u = (bq % 4) * 4
            rx = plsc.load_gather(rp_buf, [_splat(par), _splat(u)])
            ry = plsc.load_gather(rp_buf, [_splat(par), _splat(u + 1)])
            rw = plsc.load_gather(rp_buf, [_splat(par), _splat(u + 2)])
            rh = plsc.load_gather(rp_buf, [_splat(par), _splat(u + 3)])
            sclx = rw * 0.125
            scly = rh * 0.125

            for h in range(_N_HEADS):
                logits = s_buf[par, pl.ds(_EMBED + h * 16, 16)]
                mx = jnp.max(logits)
                e = jnp.exp(logits - mx)
                p = e / jnp.full((16,), jnp.sum(e))

                sx = plsc.load_gather(
                    s_buf, [_splat(par), iota * 2 + (h * 32)])
                sy = plsc.load_gather(
                    s_buf, [_splat(par), iota * 2 + (h * 32 + 1)])
                x = (rx + sx * sclx) * wp - 0.5
                y = (ry + sy * scly) * hp - 0.5
                xi = x.astype(jnp.int32)
                xf = xi.astype(jnp.float32)
                x0i = jnp.where(xf > x, xi - 1, xi)
                fx = x - jnp.where(xf > x, xf - 1.0, xf)
                yi = y.astype(jnp.int32)
                yf = yi.astype(jnp.float32)
                y0i = jnp.where(yf > y, yi - 1, yi)
                fy = y - jnp.where(yf > y, yf - 1.0, yf)
                gx = 1.0 - fx
                gy = 1.0 - fy
                x1i = x0i + 1
                y1i = y0i + 1
                vx0 = (x0i >= 0) & (x0i <= wm1)
                vx1 = (x1i >= 0) & (x1i <= wm1)
                vy0 = (y0i >= 0) & (y0i <= hm1)
                vy1 = (y1i >= 0) & (y1i <= hm1)
                cx0 = jnp.clip(x0i, 0, wm1)
                cx1 = jnp.clip(x1i, 0, wm1)
                cy0 = jnp.clip(y0i, 0, hm1)
                cy1 = jnp.clip(y1i, 0, hm1)
                r0 = offp + cy0 * wpi
                r1 = offp + cy1 * wpi
                base = vbase + (h & 3)
                corners = (
                    (r0 + cx0, gx * gy * (vx0 & vy0).astype(jnp.float32)),
                    (r0 + cx1, fx * gy * (vx1 & vy0).astype(jnp.float32)),
                    (r1 + cx0, gx * fy * (vx0 & vy1).astype(jnp.float32)),
                    (r1 + cx1, fx * fy * (vx1 & vy1).astype(jnp.float32)),
                )
                for c, (ridx, wgt) in enumerate(corners):
                    flat = h * 64 + c * 16
                    idx_buf[par, flat >> 7, pl.ds(flat & 127, 16)] = (
                        ridx * 64 + base)
                    w_buf[par, pl.ds(flat, 16)] = wgt * p

            for j in range(4):
                tbl = ve_hbm if j < 2 else vo_hbm
                pltpu.async_copy(
                    tbl.at[idx_buf.at[par, j]],
                    g_buf.at[par, pl.ds(j * 128, 128)],
                    gsem.at[par],
                )


        @pl.when(i > 0)
        def _consume():
            i1 = i - 1
            par1 = lax.rem(i1, 2)
            for _ in range(4):
                pltpu.make_async_copy(
                    ve_hbm.at[idx_buf.at[par1, 0]],
                    g_buf.at[par1, pl.ds(0, 128)],
                    gsem.at[par1],
                ).wait()

            row0 = wid * _ROW0_STEP + i1 * _N_HEADS

            @pl.when(i1 >= 2)
            def _drain_out():
                pltpu.make_async_copy(
                    out_buf.at[par1],
                    out_hbm.at[pl.ds(row0, _N_HEADS)],
                    osem.at[par1]).wait()

            def acc_row(r, carry2):
                acc0 = jnp.zeros((16,), jnp.float32)
                acc1 = jnp.zeros((16,), jnp.float32)
                for j in range(64):
                    n_ = r * 64 + j
                    wj = plsc.load_gather(
                        w_buf, [_splat(par1), _splat(n_)])
                    acc0 = acc0 + wj * g_buf[par1, n_, pl.ds(0, 16)]
                    acc1 = acc1 + wj * g_buf[par1, n_, pl.ds(16, 16)]
                out_buf[par1, r, pl.ds(0, 16)] = acc0
                out_buf[par1, r, pl.ds(16, 16)] = acc1
                return carry2

            lax.fori_loop(0, _N_HEADS, acc_row, 0)

            pltpu.async_copy(
                out_buf.at[par1],
                out_hbm.at[pl.ds(row0, _N_HEADS)],
                osem.at[par1])

        return carry

    lax.fori_loop(0, _BQ_PER_W + 1, body, 0)

    for parf in range(2):
        pltpu.make_async_copy(
            out_buf.at[parf],
            out_hbm.at[pl.ds(wid * _ROW0_STEP, _N_HEADS)],
            osem.at[parf]).wait()


def _sc_sample(s_all, rp16, veven, vodd):
    mesh = plsc.VectorSubcoreMesh(core_axis_name="c", subcore_axis_name="s")
    f = pl.kernel(
        _sc_body,
        out_type=jax.ShapeDtypeStruct((_ROWS, _HEAD_DIM), jnp.float32),
        mesh=mesh,
        compiler_params=pltpu.CompilerParams(
            needs_layout_passes=False, use_tc_tiling_on_sc=False),
        scratch_types=[
            pltpu.VMEM((2, 384), jnp.float32),
            pltpu.VMEM((2, 16), jnp.float32),
            pltpu.VMEM((2, 4, 128), jnp.int32),
            pltpu.VMEM((2, 512), jnp.float32),
            pltpu.VMEM((2, 512, _HEAD_DIM), jnp.float32),
            pltpu.VMEM((2, _N_HEADS, _HEAD_DIM), jnp.float32),
            pltpu.SemaphoreType.DMA((2,)),
            pltpu.SemaphoreType.DMA((2,)),
            pltpu.SemaphoreType.DMA((2,)),
        ],
    )
    return f(s_all, rp16, veven, vodd)


def kernel(query, reference_points, value, Wv, bv, Ws, bs_, Wa, ba, Wo, bo):
    q2 = query.reshape(_BQ, _EMBED)
    v2 = value.transpose(1, 0, 2).reshape(_LV * _B, _EMBED)

    ve, vo = _matmul_bias_split(v2, Wv, bv, bm=1088)
    wsa = jnp.concatenate([Ws, Wa], axis=1)
    bsa = jnp.concatenate([bs_, ba])
    s_all = _matmul_bias(q2, wsa, bsa, bm=600)

    rp16 = reference_points.reshape(_BQ // 4, 16)
    veven = ve.reshape(_LV * _B * 4, _HEAD_DIM)
    vodd = vo.reshape(_LV * _B * 4, _HEAD_DIM)
    sampled = _sc_sample(s_all, rp16, veven, vodd)

    out = _matmul_bias(sampled.reshape(_BQ, _EMBED), Wo, bo, bm=600)
    return out.reshape(_B, _LQ, _EMBED)
